# baseline (speedup 1.0000x reference)
import numpy as np
import jax
import jax.numpy as jnp
from jax import lax
from jax.experimental import pallas as pl
from jax.experimental.pallas import tpu as pltpu
from jax.experimental.pallas import tpu_sc as plsc

_N_SPH = 7
_N_RAD = 6
_CUTOFF = 5.0


def _jn_np(x, l):
    x = float(x)
    j0 = np.sin(x) / x
    if l == 0:
        return j0
    j1 = np.sin(x) / x ** 2 - np.cos(x) / x
    if l == 1:
        return j1
    jm1, jc = j0, j1
    for n in range(1, l):
        jm1, jc = jc, (2 * n + 1) / x * jc - jm1
    return jc


def _jn_zeros_np(n, k):
    zerosj = np.zeros((n, k))
    zerosj[0] = np.arange(1, k + 1) * np.pi
    points = np.arange(1, k + n) * np.pi
    racines = np.zeros(k + n - 1)
    for i in range(1, n):
        for j in range(k + n - 1 - i):
            a, b = points[j], points[j + 1]
            fa = _jn_np(a, i)
            for _ in range(200):
                m = 0.5 * (a + b)
                fm = _jn_np(m, i)
                if fa * fm <= 0.0:
                    b = m
                else:
                    a, fa = m, fm
            racines[j] = 0.5 * (a + b)
        points = racines.copy()
        zerosj[i][:k] = racines[:k]
    return zerosj


_ZEROS = _jn_zeros_np(_N_SPH, _N_RAD)
_NORMS = np.zeros((_N_SPH, _N_RAD))
for _l in range(_N_SPH):
    for _n in range(_N_RAD):
        _NORMS[_l, _n] = 1.0 / np.sqrt(0.5 * _jn_np(_ZEROS[_l, _n], _l + 1) ** 2)

_NCOL = _N_SPH * _N_RAD
_NPAD = 48

_ZZ = np.ones((_NPAD, 1), np.float32)
_ZZ[:_NCOL, 0] = (_ZEROS.reshape(-1).astype(np.float32)
                  * np.float32(0.2)).astype(np.float32)
_NR = np.zeros((_NPAD, 1), np.float32)
_NR[:_NCOL, 0] = _NORMS.reshape(-1).astype(np.float32)
_LROW = np.zeros((_NPAD, 1), np.int32)
_LROW[:_NCOL, 0] = np.repeat(np.arange(_N_SPH), _N_RAD)
_CBF_C = [np.sqrt((2 * l + 1) / (4.0 * np.pi)).astype(np.float32)
          for l in range(_N_SPH)]

_N_TRI = 1600000
_N_ROWS = _N_TRI // 128
_R = 64
_GRID = (_N_ROWS + _R - 1) // _R

_NW = 32
_G = 8
_ROWS_PAD = 12544
_RPW = _ROWS_PAD // _NW
_NG = _RPW // _G


def _sc_gather_kernel(dist_hbm, idx_hbm, out_hbm, idx0, idx1, val0, val1,
                      s0, s1):
    wid = lax.axis_index("s") * 2 + lax.axis_index("c")
    base = wid * _RPW

    def load_idx(buf, g):
        pltpu.sync_copy(idx_hbm.at[pl.ds(base + g * _G, _G)], buf)

    def fire(ibuf, vbuf, sem):
        for r in range(_G):
            pltpu.async_copy(dist_hbm.at[ibuf.at[r]], vbuf.at[r], sem)

    def drain(ibuf, vbuf, sem):
        for r in range(_G):
            pltpu.make_async_copy(dist_hbm.at[ibuf.at[r]], vbuf.at[r],
                                  sem).wait()

    def store(vbuf, g):
        pltpu.sync_copy(vbuf, out_hbm.at[pl.ds(base + g * _G, _G)])

    load_idx(idx0, 0)
    fire(idx0, val0, s0)

    def body(k, carry):
        ga = 2 * k + 1
        gb = 2 * k + 2
        load_idx(idx1, ga)
        fire(idx1, val1, s1)
        drain(idx0, val0, s0)
        store(val0, ga - 1)
        load_idx(idx0, gb)
        fire(idx0, val0, s0)
        drain(idx1, val1, s1)
        store(val1, gb - 1)
        return carry

    lax.fori_loop(0, (_NG - 1) // 2, body, 0)
    drain(idx0, val0, s0)
    store(val0, _NG - 1)


def _sc_gather(dist, idx2):
    mesh = plsc.VectorSubcoreMesh(core_axis_name="c", subcore_axis_name="s")
    f = pl.kernel(
        _sc_gather_kernel,
        out_type=jax.ShapeDtypeStruct((_ROWS_PAD, 128), jnp.float32),
        mesh=mesh,
        scratch_types=[
            pltpu.VMEM((_G, 128), jnp.int32),
            pltpu.VMEM((_G, 128), jnp.int32),
            pltpu.VMEM((_G, 128), jnp.float32),
            pltpu.VMEM((_G, 128), jnp.float32),
            pltpu.SemaphoreType.DMA,
            pltpu.SemaphoreType.DMA,
        ],
    )
    return f(dist, idx2)


def _tc_body(dkj_ref, ang_ref, cst_ref, out_ref):
    zz = cst_ref[:, 0:1]
    nr = cst_ref[:, 1:2]
    lrow = cst_ref[:, 2:3]
    lmask = [lrow == float(k) for k in range(_N_SPH)]
    dist_all = dkj_ref[:, :]
    d_all = dist_all * 0.2
    d2 = d_all * d_all
    d4 = d2 * d2
    d5 = d4 * d_all
    env_all = (1.0 / d_all + (-28.0) * d5 + 48.0 * d5 * d_all
               + (-21.0) * d5 * d_all * d_all)
    env_all = jnp.where(d_all < 1.0, env_all, 0.0)
    ct_all = jnp.cos(ang_ref[:, :])
    pls_all = [jnp.ones_like(ct_all), ct_all]
    for l in range(1, _N_SPH - 1):
        pls_all.append(((2 * l + 1) * ct_all * pls_all[l] - l * pls_all[l - 1])
                       / (l + 1))
    cbf_all = [_CBF_C[l] * pls_all[l] for l in range(_N_SPH)]
    for g in range(_R):
        dist_row = dist_all[g:g + 1, :]
        x = zz * dist_row
        s = jnp.sin(x)
        cth = jnp.cos(x)
        j0 = s / x
        j1 = s / (x * x) - cth / x
        jm1, jc = j0, j1
        jsel = jnp.where(lmask[0], j0, j1)
        for n in range(1, _N_SPH - 1):
            jm1, jc = jc, ((2 * n + 1) / x) * jc - jm1
            jsel = jnp.where(lmask[n + 1], jc, jsel)
        env = env_all[g:g + 1, :]
        cbf48 = jnp.concatenate(
            [jnp.broadcast_to(cbf_all[l][g:g + 1, :], (_N_RAD, 128))
             for l in range(_N_SPH)]
            + [jnp.zeros((_NPAD - _NCOL, 128), jnp.float32)], axis=0)
        val = (nr * jsel)
        val = env * val
        val = val * cbf48
        y = jnp.transpose(val, (1, 0))
        out_ref[g * 128:(g + 1) * 128, :] = y[:, :_NCOL]


_CST = np.zeros((_NPAD, 4), np.float32)
_CST[:, 0:1] = _ZZ
_CST[:, 1:2] = _NR
_CST[:, 2:3] = _LROW.astype(np.float32)


def _tc_call(dkj2, ang2, interpret=False):
    return pl.pallas_call(
        _tc_body,
        grid=(_GRID,),
        in_specs=[
            pl.BlockSpec((_R, 128), lambda i: (i, 0)),
            pl.BlockSpec((_R, 128), lambda i: (i, 0)),
            pl.BlockSpec((_NPAD, 4), lambda i: (0, 0)),
        ],
        out_specs=pl.BlockSpec((_R * 128, _NCOL), lambda i: (i, 0)),
        out_shape=jax.ShapeDtypeStruct((_N_TRI, _NCOL), jnp.float32),
        interpret=interpret,
    )(dkj2, ang2, jnp.asarray(_CST))


def kernel(dist, angle, edge_idx_kj):
    n_tri = angle.shape[0]
    idx_pad = jnp.zeros((_ROWS_PAD * 128,), jnp.int32).at[:n_tri].set(edge_idx_kj)
    dkj2 = _sc_gather(dist, idx_pad.reshape(_ROWS_PAD, 128))
    return _tc_call(dkj2, angle.reshape(_N_ROWS, 128))


# speedup vs baseline: 1.0005x; 1.0005x over previous
import numpy as np
import jax
import jax.numpy as jnp
from jax import lax
from jax.experimental import pallas as pl
from jax.experimental.pallas import tpu as pltpu
from jax.experimental.pallas import tpu_sc as plsc

_N_SPH = 7
_N_RAD = 6
_CUTOFF = 5.0


def _jn_np(x, l):
    x = float(x)
    j0 = np.sin(x) / x
    if l == 0:
        return j0
    j1 = np.sin(x) / x ** 2 - np.cos(x) / x
    if l == 1:
        return j1
    jm1, jc = j0, j1
    for n in range(1, l):
        jm1, jc = jc, (2 * n + 1) / x * jc - jm1
    return jc


def _jn_zeros_np(n, k):
    zerosj = np.zeros((n, k))
    zerosj[0] = np.arange(1, k + 1) * np.pi
    points = np.arange(1, k + n) * np.pi
    racines = np.zeros(k + n - 1)
    for i in range(1, n):
        for j in range(k + n - 1 - i):
            a, b = points[j], points[j + 1]
            fa = _jn_np(a, i)
            for _ in range(200):
                m = 0.5 * (a + b)
                fm = _jn_np(m, i)
                if fa * fm <= 0.0:
                    b = m
                else:
                    a, fa = m, fm
            racines[j] = 0.5 * (a + b)
        points = racines.copy()
        zerosj[i][:k] = racines[:k]
    return zerosj


_ZEROS = _jn_zeros_np(_N_SPH, _N_RAD)
_NORMS = np.zeros((_N_SPH, _N_RAD))
for _l in range(_N_SPH):
    for _n in range(_N_RAD):
        _NORMS[_l, _n] = 1.0 / np.sqrt(0.5 * _jn_np(_ZEROS[_l, _n], _l + 1) ** 2)

_NCOL = _N_SPH * _N_RAD
_NPAD = 48

_ZZ = np.ones((_NPAD, 1), np.float32)
_ZZ[:_NCOL, 0] = (_ZEROS.reshape(-1).astype(np.float32)
                  * np.float32(0.2)).astype(np.float32)
_NR = np.zeros((_NPAD, 1), np.float32)
_NR[:_NCOL, 0] = _NORMS.reshape(-1).astype(np.float32)
_LROW = np.zeros((_NPAD, 1), np.int32)
_LROW[:_NCOL, 0] = np.repeat(np.arange(_N_SPH), _N_RAD)
_CBF_C = [np.sqrt((2 * l + 1) / (4.0 * np.pi)).astype(np.float32)
          for l in range(_N_SPH)]

_N_TRI = 1600000
_N_ROWS = _N_TRI // 128
_R = 64
_GRID = (_N_ROWS + _R - 1) // _R

_NW = 32
_G = 8
_ROWS_PAD = 12544
_RPW = _ROWS_PAD // _NW
_NG = _RPW // _G


def _sc_gather_kernel(dist_hbm, idx_hbm, out_hbm, idx0, idx1, val0, val1,
                      s0, s1):
    wid = lax.axis_index("s") * 2 + lax.axis_index("c")
    base = wid * _RPW

    def load_idx(buf, g):
        pltpu.sync_copy(idx_hbm.at[pl.ds(base + g * _G, _G)], buf)

    def fire(ibuf, vbuf, sem):
        for r in range(_G):
            pltpu.async_copy(dist_hbm.at[ibuf.at[r]], vbuf.at[r], sem)

    def drain(ibuf, vbuf, sem):
        for r in range(_G):
            pltpu.make_async_copy(dist_hbm.at[ibuf.at[r]], vbuf.at[r],
                                  sem).wait()

    def store(vbuf, g):
        pltpu.sync_copy(vbuf, out_hbm.at[pl.ds(base + g * _G, _G)])

    load_idx(idx0, 0)
    fire(idx0, val0, s0)

    def body(k, carry):
        ga = 2 * k + 1
        gb = 2 * k + 2
        load_idx(idx1, ga)
        fire(idx1, val1, s1)
        drain(idx0, val0, s0)
        store(val0, ga - 1)
        load_idx(idx0, gb)
        fire(idx0, val0, s0)
        drain(idx1, val1, s1)
        store(val1, gb - 1)
        return carry

    lax.fori_loop(0, (_NG - 1) // 2, body, 0)
    drain(idx0, val0, s0)
    store(val0, _NG - 1)


def _sc_gather(dist, idx2):
    mesh = plsc.VectorSubcoreMesh(core_axis_name="c", subcore_axis_name="s")
    f = pl.kernel(
        _sc_gather_kernel,
        out_type=jax.ShapeDtypeStruct((_ROWS_PAD, 128), jnp.float32),
        mesh=mesh,
        scratch_types=[
            pltpu.VMEM((_G, 128), jnp.int32),
            pltpu.VMEM((_G, 128), jnp.int32),
            pltpu.VMEM((_G, 128), jnp.float32),
            pltpu.VMEM((_G, 128), jnp.float32),
            pltpu.SemaphoreType.DMA,
            pltpu.SemaphoreType.DMA,
        ],
    )
    return f(dist, idx2)


def _tc_body(dkj_ref, ang_ref, cst_ref, out_ref):
    zz = cst_ref[:, 0:1]
    nr = cst_ref[:, 1:2]
    lrow = cst_ref[:, 2:3]
    lmask = [lrow == float(k) for k in range(_N_SPH)]
    dist_all = dkj_ref[:, :]
    d_all = dist_all * 0.2
    d2 = d_all * d_all
    d4 = d2 * d2
    d5 = d4 * d_all
    env_all = (1.0 / d_all + (-28.0) * d5 + 48.0 * d5 * d_all
               + (-21.0) * d5 * d_all * d_all)
    env_all = jnp.where(d_all < 1.0, env_all, 0.0)
    ct_all = jnp.cos(ang_ref[:, :])
    pls_all = [jnp.ones_like(ct_all), ct_all]
    for l in range(1, _N_SPH - 1):
        pls_all.append(((2 * l + 1) * ct_all * pls_all[l] - l * pls_all[l - 1])
                       / (l + 1))
    cbf_all = [_CBF_C[l] * pls_all[l] for l in range(_N_SPH)]
    for g in range(_R):
        dist_row = dist_all[g:g + 1, :]
        x = zz * dist_row
        s = jnp.sin(x)
        cth = jnp.cos(x)
        j0 = s / x
        j1 = s / (x * x) - cth / x
        jm1, jc = j0, j1
        jsel = jnp.where(lmask[0], j0, j1)
        for n in range(1, _N_SPH - 1):
            jm1, jc = jc, ((2 * n + 1) / x) * jc - jm1
            jsel = jnp.where(lmask[n + 1], jc, jsel)
        env = env_all[g:g + 1, :]
        cbf48 = jnp.concatenate(
            [jnp.broadcast_to(cbf_all[l][g:g + 1, :], (_N_RAD, 128))
             for l in range(_N_SPH)]
            + [jnp.zeros((_NPAD - _NCOL, 128), jnp.float32)], axis=0)
        val = (nr * jsel)
        val = env * val
        val = val * cbf48
        y = jnp.transpose(val, (1, 0))
        out_ref[g * 128:(g + 1) * 128, :] = y[:, :_NCOL]


_CST = np.zeros((_NPAD, 4), np.float32)
_CST[:, 0:1] = _ZZ
_CST[:, 1:2] = _NR
_CST[:, 2:3] = _LROW.astype(np.float32)


def _tc_call(dkj2, ang2):
    return pl.pallas_call(
        _tc_body,
        grid=(_GRID,),
        in_specs=[
            pl.BlockSpec((_R, 128), lambda i: (i, 0)),
            pl.BlockSpec((_R, 128), lambda i: (i, 0)),
            pl.BlockSpec((_NPAD, 4), lambda i: (0, 0)),
        ],
        out_specs=pl.BlockSpec((_R * 128, _NCOL), lambda i: (i, 0)),
        out_shape=jax.ShapeDtypeStruct((_N_TRI, _NCOL), jnp.float32),
    )(dkj2, ang2, jnp.asarray(_CST))


def kernel(dist, angle, edge_idx_kj):
    n_tri = angle.shape[0]
    idx_pad = jnp.zeros((_ROWS_PAD * 128,), jnp.int32).at[:n_tri].set(edge_idx_kj)
    dkj2 = _sc_gather(dist, idx_pad.reshape(_ROWS_PAD, 128))
    return _tc_call(dkj2, angle.reshape(_N_ROWS, 128))
